# Initial kernel scaffold; baseline (speedup 1.0000x reference)
import functools

import jax
import jax.numpy as jnp
from jax import lax
from jax.experimental import pallas as pl
from jax.experimental.pallas import tpu as pltpu
from jax.experimental.pallas import tpu_sc as plsc

N = 10000
E = 320000
D = 128

NTILES = 32
NSUB = 16
NP = 10016
RPT = NP // NSUB
CHW = 128
NCHUNK = 2528
EP = NCHUNK * CHW
CPT = NCHUNK // NTILES

_mesh = plsc.VectorSubcoreMesh(core_axis_name="c", subcore_axis_name="s")


@functools.partial(
    pl.kernel,
    mesh=_mesh,
    out_type=jax.ShapeDtypeStruct((2, NP, D), jnp.float32),
    scratch_types=[
        pltpu.VMEM((CPT, CHW), jnp.int32),
        pltpu.VMEM((CPT, CHW), jnp.int32),
        pltpu.VMEM((CHW, D), jnp.float32),
        pltpu.VMEM_SHARED((NP, D), jnp.float32),
    ],
)
def _segsum(table_hbm, src_hbm, dst_hbm, zeros_hbm, out_hbm,
            src_v, dst_v, rows_v, acc):
    cid = lax.axis_index("c")
    sid = lax.axis_index("s")
    wid = sid * 2 + cid
    pltpu.sync_copy(zeros_hbm, acc.at[pl.ds(sid * RPT, RPT)])
    pltpu.sync_copy(src_hbm.at[wid], src_v)
    pltpu.sync_copy(dst_hbm.at[wid], dst_v)
    plsc.subcore_barrier()

    def body(j, carry):
        pltpu.sync_copy(table_hbm.at[src_v.at[j]], rows_v)
        pltpu.sync_copy(rows_v, acc.at[dst_v.at[j]], add=True)
        return carry

    lax.fori_loop(0, CPT, body, 0)
    plsc.subcore_barrier()
    pltpu.sync_copy(acc.at[pl.ds(sid * RPT, RPT)],
                    out_hbm.at[cid, pl.ds(sid * RPT, RPT)])


def _edge_segsum(table, src3, dst3, zeros):
    parts = _segsum(table, src3, dst3, zeros)
    return parts[0, :N] + parts[1, :N]


def kernel(x, pe, edge_index, W_enc, b_enc, W0, b0, g0, be0,
           W1, b1, g1, be1, W2, b2, W_dec, b_dec):
    src = edge_index[0]
    dst = edge_index[1]

    pad = EP - E
    src3 = jnp.concatenate([src, jnp.full((pad,), N, jnp.int32)]).reshape(
        NTILES, CPT, CHW)
    dst3 = jnp.concatenate([dst, jnp.full((pad,), N, jnp.int32)]).reshape(
        NTILES, CPT, CHW)
    zeros = jnp.zeros((RPT, D), jnp.float32)

    deg = jax.ops.segment_sum(jnp.ones((E,), jnp.float32), dst,
                              num_segments=N) + 1.0
    dinv = (1.0 / jnp.sqrt(deg))[:, None]

    h = jnp.concatenate([x, pe], axis=-1) @ W_enc + b_enc

    def conv(h, W, b):
        hs = dinv * (h @ W)
        hsp = jnp.pad(hs, ((0, NP - N), (0, 0)))
        agg = _edge_segsum(hsp, src3, dst3, zeros)
        return dinv * (agg + hs) + b

    def bn_relu(h, g, be):
        mu = jnp.mean(h, axis=0)
        var = jnp.var(h, axis=0)
        return jax.nn.relu((h - mu) / jnp.sqrt(var + 1e-5) * g + be)

    h = bn_relu(conv(h, W0, b0), g0, be0)
    h = bn_relu(conv(h, W1, b1), g1, be1)
    h = conv(h, W2, b2)
    return h @ W_dec + b_dec


# speedup vs baseline: 8.3861x; 8.3861x over previous
import functools

import jax
import jax.numpy as jnp
from jax import lax
from jax.experimental import pallas as pl
from jax.experimental.pallas import tpu as pltpu
from jax.experimental.pallas import tpu_sc as plsc

N = 10000
E = 320000
D = 128

NTILES = 32
NSUB = 16
NP = 10112
RPT = NP // NSUB
CHW = 128
NCHUNK = 2528
EP = NCHUNK * CHW
CPT = NCHUNK // NTILES

_mesh = plsc.VectorSubcoreMesh(core_axis_name="c", subcore_axis_name="s")


@functools.partial(
    pl.kernel,
    mesh=_mesh,
    out_type=jax.ShapeDtypeStruct((2, NP, D), jnp.float32),
    scratch_types=[
        pltpu.VMEM((CPT, CHW), jnp.int32),
        pltpu.VMEM((CPT, CHW), jnp.int32),
        pltpu.VMEM((CHW, D), jnp.float32),
        pltpu.VMEM_SHARED((NP, D), jnp.float32),
    ],
)
def _segsum(table_hbm, src_hbm, dst_hbm, zeros_hbm, out_hbm,
            src_v, dst_v, rows_v, acc):
    cid = lax.axis_index("c")
    sid = lax.axis_index("s")
    wid = sid * 2 + cid
    pltpu.sync_copy(zeros_hbm, acc.at[pl.ds(sid * RPT, RPT)])
    pltpu.sync_copy(src_hbm.at[wid], src_v)
    pltpu.sync_copy(dst_hbm.at[wid], dst_v)
    plsc.subcore_barrier()

    def body(j, carry):
        pltpu.sync_copy(table_hbm.at[src_v.at[j]], rows_v)
        pltpu.sync_copy(rows_v, acc.at[dst_v.at[j]], add=True)
        return carry

    lax.fori_loop(0, CPT, body, 0)
    plsc.subcore_barrier()
    pltpu.sync_copy(acc.at[pl.ds(sid * RPT, RPT)],
                    out_hbm.at[cid, pl.ds(sid * RPT, RPT)])


def _edge_segsum(table, src3, dst3, zeros):
    parts = _segsum(table, src3, dst3, zeros)
    return parts[0, :N] + parts[1, :N]


def kernel(x, pe, edge_index, W_enc, b_enc, W0, b0, g0, be0,
           W1, b1, g1, be1, W2, b2, W_dec, b_dec):
    src = edge_index[0]
    dst = edge_index[1]

    pad = EP - E
    src3 = jnp.concatenate([src, jnp.full((pad,), N, jnp.int32)]).reshape(
        NTILES, CPT, CHW)
    dst3 = jnp.concatenate([dst, jnp.full((pad,), N, jnp.int32)]).reshape(
        NTILES, CPT, CHW)
    zeros = jnp.zeros((RPT, D), jnp.float32)

    deg = jax.ops.segment_sum(jnp.ones((E,), jnp.float32), dst,
                              num_segments=N) + 1.0
    dinv = (1.0 / jnp.sqrt(deg))[:, None]

    h = jnp.concatenate([x, pe], axis=-1) @ W_enc + b_enc

    def conv(h, W, b):
        hs = dinv * (h @ W)
        hsp = jnp.pad(hs, ((0, NP - N), (0, 0)))
        agg = _edge_segsum(hsp, src3, dst3, zeros)
        return dinv * (agg + hs) + b

    def bn_relu(h, g, be):
        mu = jnp.mean(h, axis=0)
        var = jnp.var(h, axis=0)
        return jax.nn.relu((h - mu) / jnp.sqrt(var + 1e-5) * g + be)

    h = bn_relu(conv(h, W0, b0), g0, be0)
    h = bn_relu(conv(h, W1, b1), g1, be1)
    h = conv(h, W2, b2)
    return h @ W_dec + b_dec
